# Initial kernel scaffold; baseline (speedup 1.0000x reference)
import jax
import jax.numpy as jnp
from jax.experimental import pallas as pl


def kernel(user_emb, item_emb, adj_indices, adj_values, users, pos_items, neg_items):
    raise NotImplementedError("write your pallas kernel here")


# speedup vs baseline: 7.4838x; 7.4838x over previous
import functools

import jax
import jax.numpy as jnp
from jax import lax
from jax.experimental import pallas as pl
from jax.experimental.pallas import tpu as pltpu
from jax.experimental.pallas import tpu_sc as plsc

N_USER = 10000
N_ITEM = 40000
N = N_USER + N_ITEM
E = 800000
D = 64
B = 4096
N_LAYERS = 3

H = D // 2
NS = 16
EP = E // NS
C = 400
KB = 80
J = C // KB
NITER = EP // C
NCHUNK = NS * NITER
B3 = 3 * B
RP = B3 // NS
RKB = 96
RJ = RP // RKB
NPAD = 51200
ZPT = NPAD // NS
KZ = ZPT // C


def _sc_body(ego0, ego1, metar, idxr, out0, out1,
             acc, meta_v, rows_v, idx_v, erows_v, srows_v, sem_g, sem_s):
  cid = lax.axis_index("c")
  sid = lax.axis_index("s")

  zero16 = jnp.zeros((16,), jnp.float32)

  def zbody(r, carry):
    rows_v[r, pl.ds(0, 16)] = zero16
    rows_v[r, pl.ds(16, 16)] = zero16
    return carry

  lax.fori_loop(0, C, zbody, 0)
  for k in range(KZ):
    pltpu.sync_copy(rows_v, acc.at[pl.ds(sid * ZPT + k * C, C)])
  plsc.subcore_barrier()

  def accumulate(ego_h):
    def it_body(it, carry):
      chunk = sid * NITER + it
      pltpu.sync_copy(metar.at[chunk], meta_v)
      descs = []
      for j in range(J):
        descs.append(pltpu.async_copy(
            ego_h.at[meta_v.at[1, j]], rows_v.at[pl.ds(j * KB, KB)], sem_g))
      for d in descs:
        d.wait()

      def grp_body(g, carry2):
        v16 = plsc.bitcast(
            meta_v[2, g // (KB // 16), pl.ds((g % (KB // 16)) * 16, 16)],
            jnp.float32)
        for i in range(16):
          r = g * 16 + i
          bc = v16.at[jnp.full((16,), i, jnp.int32)].get(
              mode="promise_in_bounds")
          rows_v[r, pl.ds(0, 16)] = rows_v[r, pl.ds(0, 16)] * bc
          rows_v[r, pl.ds(16, 16)] = rows_v[r, pl.ds(16, 16)] * bc
        return carry2

      lax.fori_loop(0, C // 16, grp_body, 0)

      descs = []
      for j in range(J):
        descs.append(pltpu.async_copy(
            rows_v.at[pl.ds(j * KB, KB)], acc.at[meta_v.at[0, j]], sem_s,
            add=True))
      for d in descs:
        d.wait()
      return carry

    lax.fori_loop(0, NITER, it_body, 0)

  pl.when(cid == 0)(lambda: accumulate(ego0))
  pl.when(cid == 1)(lambda: accumulate(ego1))
  plsc.subcore_barrier()

  pltpu.sync_copy(idxr.at[pl.ds(sid * RJ, RJ)], idx_v)

  def readout(ego_h, out_h):
    for j in range(RJ):
      dg = pltpu.async_copy(ego_h.at[idx_v.at[j]], erows_v, sem_g)
      ds = pltpu.async_copy(acc.at[idx_v.at[j]], srows_v, sem_s)
      dg.wait()
      ds.wait()

      def cbody(r, carry):
        for lo in (0, 16):
          e = erows_v[r, pl.ds(lo, 16)]
          s = srows_v[r, pl.ds(lo, 16)]
          erows_v[r, pl.ds(lo, 16)] = e * 0.25 + s * 0.75
        return carry

      lax.fori_loop(0, RKB, cbody, 0)
      pltpu.sync_copy(erows_v, out_h.at[pl.ds(sid * RP + j * RKB, RKB)])

  pl.when(cid == 0)(lambda: readout(ego0, out0))
  pl.when(cid == 1)(lambda: readout(ego1, out1))


_sc_call = functools.partial(
    pl.kernel,
    mesh=plsc.VectorSubcoreMesh(core_axis_name="c", subcore_axis_name="s"),
    compiler_params=pltpu.CompilerParams(
        use_tc_tiling_on_sc=False, needs_layout_passes=False),
    out_type=(
        jax.ShapeDtypeStruct((B3, H), jnp.float32),
        jax.ShapeDtypeStruct((B3, H), jnp.float32),
    ),
    scratch_types=[
        pltpu.VMEM_SHARED((NPAD, H), jnp.float32),
        pltpu.VMEM((3, J, KB), jnp.int32),
        pltpu.VMEM((C, H), jnp.float32),
        pltpu.VMEM((RJ, RKB), jnp.int32),
        pltpu.VMEM((RKB, H), jnp.float32),
        pltpu.VMEM((RKB, H), jnp.float32),
        pltpu.SemaphoreType.DMA,
        pltpu.SemaphoreType.DMA,
    ],
)(_sc_body)


def kernel(user_emb, item_emb, adj_indices, adj_values, users, pos_items,
           neg_items):
  ego = jnp.concatenate([user_emb, item_emb], axis=0)
  ego0 = ego[:, :H]
  ego1 = ego[:, H:]
  dst = adj_indices[0].astype(jnp.int32).reshape(NCHUNK, J, KB)
  src = adj_indices[1].astype(jnp.int32).reshape(NCHUNK, J, KB)
  val = jax.lax.bitcast_convert_type(
      adj_values.astype(jnp.float32), jnp.int32).reshape(NCHUNK, J, KB)
  meta = jnp.stack([dst, src, val], axis=1)
  idx_all = jnp.concatenate([
      users.astype(jnp.int32),
      pos_items.astype(jnp.int32) + N_USER,
      neg_items.astype(jnp.int32) + N_USER,
  ]).reshape(B3 // RKB, RKB)
  out0, out1 = _sc_call(ego0, ego1, meta, idx_all)
  full = jnp.concatenate([out0, out1], axis=1)
  return (full[:B], full[B:2 * B], full[2 * B:])


# speedup vs baseline: 12.9274x; 1.7274x over previous
import functools

import jax
import jax.numpy as jnp
from jax import lax
from jax.experimental import pallas as pl
from jax.experimental.pallas import tpu as pltpu
from jax.experimental.pallas import tpu_sc as plsc

N_USER = 10000
N_ITEM = 40000
N = N_USER + N_ITEM
E = 800000
D = 64
B = 4096
N_LAYERS = 3

H = D // 2
NS = 16
EP = E // NS
C = 400
KB = 80
J = C // KB
G = KB // 16
NITER = EP // C
NCHUNK = NS * NITER
RKB = 128
RPT = B // NS
RJ = RPT // RKB
NPAD = 50176
ZPT = NPAD // NS
ZROWS = 392


def _scale16(vec_ref, row, v16):
  pass


def _sc_body(ego2, dstr, srcr, valr, idxu, idxp, idxn, outu, outp, outn,
             acc, dst_a, dst_b, src_a, src_b, val_a, val_b, src2_a, src2_b,
             rows_a, rows_b, idx_v, idx2_v, sem_m, sem_g, sem_s):
  cid = lax.axis_index("c")
  sid = lax.axis_index("s")

  zero16 = jnp.zeros((16,), jnp.float32)

  def zbody(r, carry):
    rows_a[r, pl.ds(0, 16)] = zero16
    rows_a[r, pl.ds(16, 16)] = zero16
    return carry

  lax.fori_loop(0, ZROWS, zbody, 0)
  for k in range(ZPT // ZROWS):
    pltpu.sync_copy(rows_a.at[pl.ds(0, ZROWS)],
                    acc.at[pl.ds(sid * ZPT + k * ZROWS, ZROWS)])
  plsc.subcore_barrier()

  def fetch_meta(it, dstb, srcb, valb):
    chunk = sid * NITER + it
    return [pltpu.async_copy(dstr.at[chunk], dstb, sem_m),
            pltpu.async_copy(srcr.at[chunk], srcb, sem_m),
            pltpu.async_copy(valr.at[chunk], valb, sem_m)]

  def compute_src2(srcb, src2b):
    cvec = jnp.full((16,), 0, jnp.int32) + cid

    def sbody(g, carry):
      s16 = srcb[g // G, pl.ds((g % G) * 16, 16)]
      src2b[g // G, pl.ds((g % G) * 16, 16)] = s16 + s16 + cvec
      return carry

    lax.fori_loop(0, C // 16, sbody, 0)

  def fire_gather(src2b, rowsb):
    return [pltpu.async_copy(
        ego2.at[src2b.at[j]], rowsb.at[pl.ds(j * KB, KB)], sem_g)
            for j in range(J)]

  def scale(valb, rowsb):
    def grp_body(g, carry):
      v16 = valb[g // G, pl.ds((g % G) * 16, 16)]
      for i in range(16):
        r = g * 16 + i
        bc = v16.at[jnp.full((16,), i, jnp.int32)].get(
            mode="promise_in_bounds")
        rowsb[r, pl.ds(0, 16)] = rowsb[r, pl.ds(0, 16)] * bc
        rowsb[r, pl.ds(16, 16)] = rowsb[r, pl.ds(16, 16)] * bc
      return carry

    lax.fori_loop(0, C // 16, grp_body, 0)

  def scatter(dstb, rowsb):
    descs = [pltpu.async_copy(
        rowsb.at[pl.ds(j * KB, KB)], acc.at[dstb.at[j]], sem_s, add=True)
             for j in range(J)]
    for d in descs:
      d.wait()

  bufs = ((dst_a, src_a, val_a, src2_a, rows_a),
          (dst_b, src_b, val_b, src2_b, rows_b))

  for d in fetch_meta(0, dst_a, src_a, val_a):
    d.wait()
  compute_src2(src_a, src2_a)
  gather_descs = {0: fire_gather(src2_a, rows_a)}
  meta_descs = {1: fetch_meta(1, dst_b, src_b, val_b)}

  def pipe_body(it2, carry):
    for parity in (0, 1):
      dstb, srcb, valb, src2b, rowsb = bufs[parity]
      dstn, srcn, valn, src2n, rowsn = bufs[1 - parity]
      it = it2 * 2 + parity
      @pl.when(it + 1 < NITER)
      def _():
        pltpu.make_async_copy(dstr.at[0], dstn, sem_m).wait()
        pltpu.make_async_copy(srcr.at[0], srcn, sem_m).wait()
        pltpu.make_async_copy(valr.at[0], valn, sem_m).wait()
        compute_src2(srcn, src2n)
        for j in range(J):
          pltpu.async_copy(
              ego2.at[src2n.at[j]], rowsn.at[pl.ds(j * KB, KB)], sem_g)
      for j in range(J):
        pltpu.make_async_copy(
            ego2.at[src2b.at[j]], rowsb.at[pl.ds(j * KB, KB)], sem_g).wait()
      scale(valb, rowsb)
      scatter(dstb, rowsb)
      @pl.when(it + 2 < NITER)
      def _():
        fetch_meta(it + 2, dstb, srcb, valb)
    return carry

  lax.fori_loop(0, NITER // 2, pipe_body, 0)
  for j in range(J):
    pltpu.make_async_copy(
        ego2.at[src2_a.at[j]], rows_a.at[pl.ds(j * KB, KB)], sem_g).wait()
  scale(val_a, rows_a)
  scatter(dst_a, rows_a)
  plsc.subcore_barrier()

  def readout(idxr, outr):
    for jj in range(RJ):
      row = sid * RJ + jj
      pltpu.sync_copy(idxr.at[pl.ds(row, 1)], idx_v)
      cvec = jnp.full((16,), 0, jnp.int32) + cid

      def ibody(g, carry):
        s16 = idx_v[0, pl.ds(g * 16, 16)]
        idx2_v[0, pl.ds(g * 16, 16)] = s16 + s16 + cvec
        return carry

      lax.fori_loop(0, RKB // 16, ibody, 0)
      dg = pltpu.async_copy(ego2.at[idx2_v.at[0]],
                            rows_a.at[pl.ds(0, RKB)], sem_g)
      dsde = pltpu.async_copy(acc.at[idx_v.at[0]],
                              rows_a.at[pl.ds(RKB, RKB)], sem_s)
      dg.wait()
      dsde.wait()

      def cbody(r, carry):
        for lo in (0, 16):
          e = rows_a[r, pl.ds(lo, 16)]
          s = rows_a[RKB + r, pl.ds(lo, 16)]
          rows_a[r, pl.ds(lo, 16)] = e * 0.25 + s * 0.75
        return carry

      lax.fori_loop(0, RKB, cbody, 0)
      pltpu.sync_copy(
          rows_a.at[pl.ds(0, RKB)],
          outr.at[pl.ds(sid * RPT + jj * RKB, RKB), pl.ds(cid * H, H)])

  readout(idxu, outu)
  readout(idxp, outp)
  readout(idxn, outn)


_sc_call = functools.partial(
    pl.kernel,
    mesh=plsc.VectorSubcoreMesh(core_axis_name="c", subcore_axis_name="s"),
    compiler_params=pltpu.CompilerParams(
        use_tc_tiling_on_sc=False, needs_layout_passes=False),
    out_type=(
        jax.ShapeDtypeStruct((B, D), jnp.float32),
        jax.ShapeDtypeStruct((B, D), jnp.float32),
        jax.ShapeDtypeStruct((B, D), jnp.float32),
    ),
    scratch_types=[
        pltpu.VMEM_SHARED((NPAD, H), jnp.float32),
        pltpu.VMEM((J, KB), jnp.int32),
        pltpu.VMEM((J, KB), jnp.int32),
        pltpu.VMEM((J, KB), jnp.int32),
        pltpu.VMEM((J, KB), jnp.int32),
        pltpu.VMEM((J, KB), jnp.float32),
        pltpu.VMEM((J, KB), jnp.float32),
        pltpu.VMEM((J, KB), jnp.int32),
        pltpu.VMEM((J, KB), jnp.int32),
        pltpu.VMEM((C, H), jnp.float32),
        pltpu.VMEM((C, H), jnp.float32),
        pltpu.VMEM((1, RKB), jnp.int32),
        pltpu.VMEM((1, RKB), jnp.int32),
        pltpu.SemaphoreType.DMA,
        pltpu.SemaphoreType.DMA,
        pltpu.SemaphoreType.DMA,
    ],
)(_sc_body)


def kernel(user_emb, item_emb, adj_indices, adj_values, users, pos_items,
           neg_items):
  ego2 = jnp.concatenate([user_emb, item_emb], axis=0).reshape(2 * N, H)
  dst = adj_indices[0].astype(jnp.int32).reshape(NCHUNK, J, KB)
  src = adj_indices[1].astype(jnp.int32).reshape(NCHUNK, J, KB)
  val = adj_values.astype(jnp.float32).reshape(NCHUNK, J, KB)
  idxu = users.astype(jnp.int32).reshape(B // RKB, RKB)
  idxp = (pos_items.astype(jnp.int32) + N_USER).reshape(B // RKB, RKB)
  idxn = (neg_items.astype(jnp.int32) + N_USER).reshape(B // RKB, RKB)
  return _sc_call(ego2, dst, src, val, idxu, idxp, idxn)


# speedup vs baseline: 16.6764x; 1.2900x over previous
import functools

import jax
import jax.numpy as jnp
from jax import lax
from jax.experimental import pallas as pl
from jax.experimental.pallas import tpu as pltpu
from jax.experimental.pallas import tpu_sc as plsc

N_USER = 10000
N_ITEM = 40000
N = N_USER + N_ITEM
E = 800000
D = 64
B = 4096
N_LAYERS = 3

H = D // 2
NS = 16
EP = E // NS
C = 400
KB = 80
J = C // KB
G = KB // 16
NITER = EP // C
NCHUNK = NS * NITER
RKB = 128
RPT = B // NS
RJ = RPT // RKB


def _sc_body(ego2, adjr, valr, idxu, idxp, idxn, outu, outp, outn,
             acc, dst0, dst1, dst2, dst3, src_a, src_b, val_a, val_b,
             rows_a, rows_b, idx_v, idx2_v, sem_m, sem_g, sem_s):
  cid = lax.axis_index("c")
  sid = lax.axis_index("s")
  dst_ring = (dst0, dst1, dst2, dst3)
  src_pp = (src_a, src_b)
  val_pp = (val_a, val_b)
  rows_pp = (rows_a, rows_b)

  zero16 = jnp.zeros((16,), jnp.float32)

  def zbody(r, carry):
    rows_a[r, pl.ds(0, 16)] = zero16
    rows_a[r, pl.ds(16, 16)] = zero16
    return carry

  lax.fori_loop(0, C, zbody, 0)
  zleft = N // NS
  for k in range(zleft // C):
    pltpu.sync_copy(rows_a, acc.at[pl.ds(sid * zleft + k * C, C)])
  pltpu.sync_copy(rows_a.at[pl.ds(0, zleft % C)],
                  acc.at[pl.ds(sid * zleft + (zleft // C) * C, zleft % C)])
  plsc.subcore_barrier()

  def fetch_meta(it, r4, p):
    chunk = sid * NITER + it
    pltpu.async_copy(adjr.at[0, chunk], dst_ring[r4], sem_m)
    pltpu.async_copy(adjr.at[1, chunk], src_pp[p], sem_m)
    pltpu.async_copy(valr.at[pl.ds(chunk * C, C)], val_pp[p], sem_m)

  def wait_meta(r4, p):
    pltpu.make_async_copy(adjr.at[0, 0], dst_ring[r4], sem_m).wait()
    pltpu.make_async_copy(adjr.at[1, 0], src_pp[p], sem_m).wait()
    pltpu.make_async_copy(valr.at[pl.ds(0, C)], val_pp[p], sem_m).wait()

  def compute_src2(p):
    srcb = src_pp[p]
    cvec = jnp.full((16,), 0, jnp.int32) + cid

    def sbody(g, carry):
      s16 = srcb[g // G, pl.ds((g % G) * 16, 16)]
      srcb[g // G, pl.ds((g % G) * 16, 16)] = s16 + s16 + cvec
      return carry

    lax.fori_loop(0, C // 16, sbody, 0)

  def fire_gather(p):
    for j in range(J):
      pltpu.async_copy(
          ego2.at[src_pp[p].at[j]], rows_pp[p].at[pl.ds(j * KB, KB)], sem_g)

  def wait_gather(p):
    for j in range(J):
      pltpu.make_async_copy(
          ego2.at[src_pp[p].at[j]],
          rows_pp[p].at[pl.ds(j * KB, KB)], sem_g).wait()

  def scale(p):
    valb, rowsb = val_pp[p], rows_pp[p]

    def grp_body(g, carry):
      v16 = valb[pl.ds(g * 16, 16)]
      for i in range(16):
        r = g * 16 + i
        bc = v16.at[jnp.full((16,), i, jnp.int32)].get(
            mode="promise_in_bounds")
        rowsb[r, pl.ds(0, 16)] = rowsb[r, pl.ds(0, 16)] * bc
        rowsb[r, pl.ds(16, 16)] = rowsb[r, pl.ds(16, 16)] * bc
      return carry

    lax.fori_loop(0, C // 16, grp_body, 0)

  def fire_scatter(r4, p):
    for j in range(J):
      pltpu.async_copy(
          rows_pp[p].at[pl.ds(j * KB, KB)], acc.at[dst_ring[r4].at[j]],
          sem_s, add=True)

  def wait_scatter():
    for j in range(J):
      pltpu.make_async_copy(
          ego2.at[pl.ds(0, KB)], acc.at[pl.ds(0, KB)], sem_s).wait()

  fetch_meta(0, 0, 0)
  fetch_meta(1, 1, 1)
  wait_meta(0, 0)
  compute_src2(0)
  fire_gather(0)

  def pipe_body(it4, carry):
    for quad in range(4):
      it = it4 * 4 + quad
      r4 = quad
      p = quad % 2
      @pl.when(it > 0)
      def _():
        wait_scatter()
      @pl.when(it + 1 < NITER)
      def _():
        wait_meta((r4 + 1) % 4, 1 - p)
        compute_src2(1 - p)
        fire_gather(1 - p)
      wait_gather(p)
      scale(p)
      fire_scatter(r4, p)
      @pl.when(it + 2 < NITER)
      def _():
        fetch_meta(it + 2, (r4 + 2) % 4, p)
    return carry

  lax.fori_loop(0, NITER // 4, pipe_body, 0)
  wait_scatter()
  wait_gather(0)
  scale(0)
  fire_scatter(0, 0)
  wait_scatter()
  plsc.subcore_barrier()

  def readout(idxr, outr):
    for jj in range(RJ):
      row = sid * RJ + jj
      pltpu.sync_copy(idxr.at[pl.ds(row, 1)], idx_v)
      cvec = jnp.full((16,), 0, jnp.int32) + cid

      def ibody(g, carry):
        s16 = idx_v[0, pl.ds(g * 16, 16)]
        idx2_v[0, pl.ds(g * 16, 16)] = s16 + s16 + cvec
        return carry

      lax.fori_loop(0, RKB // 16, ibody, 0)
      dg = pltpu.async_copy(ego2.at[idx2_v.at[0]],
                            rows_a.at[pl.ds(0, RKB)], sem_g)
      dsde = pltpu.async_copy(acc.at[idx_v.at[0]],
                              rows_a.at[pl.ds(RKB, RKB)], sem_s)
      dg.wait()
      dsde.wait()

      def cbody(r, carry):
        for lo in (0, 16):
          e = rows_a[r, pl.ds(lo, 16)]
          s = rows_a[RKB + r, pl.ds(lo, 16)]
          rows_a[r, pl.ds(lo, 16)] = e * 0.25 + s * 0.75
        return carry

      lax.fori_loop(0, RKB, cbody, 0)
      pltpu.sync_copy(
          rows_a.at[pl.ds(0, RKB)],
          outr.at[pl.ds(sid * RPT + jj * RKB, RKB), pl.ds(cid * H, H)])

  readout(idxu, outu)
  readout(idxp, outp)
  readout(idxn, outn)


_sc_call = functools.partial(
    pl.kernel,
    mesh=plsc.VectorSubcoreMesh(core_axis_name="c", subcore_axis_name="s"),
    compiler_params=pltpu.CompilerParams(
        use_tc_tiling_on_sc=False, needs_layout_passes=False),
    out_type=(
        jax.ShapeDtypeStruct((B, D), jnp.float32),
        jax.ShapeDtypeStruct((B, D), jnp.float32),
        jax.ShapeDtypeStruct((B, D), jnp.float32),
    ),
    scratch_types=[
        pltpu.VMEM_SHARED((N, H), jnp.float32),
        pltpu.VMEM((J, KB), jnp.int32),
        pltpu.VMEM((J, KB), jnp.int32),
        pltpu.VMEM((J, KB), jnp.int32),
        pltpu.VMEM((J, KB), jnp.int32),
        pltpu.VMEM((J, KB), jnp.int32),
        pltpu.VMEM((J, KB), jnp.int32),
        pltpu.VMEM((C,), jnp.float32),
        pltpu.VMEM((C,), jnp.float32),
        pltpu.VMEM((C, H), jnp.float32),
        pltpu.VMEM((C, H), jnp.float32),
        pltpu.VMEM((1, RKB), jnp.int32),
        pltpu.VMEM((1, RKB), jnp.int32),
        pltpu.SemaphoreType.DMA,
        pltpu.SemaphoreType.DMA,
        pltpu.SemaphoreType.DMA,
    ],
)(_sc_body)


def kernel(user_emb, item_emb, adj_indices, adj_values, users, pos_items,
           neg_items):
  ego2 = jnp.concatenate([user_emb, item_emb], axis=0).reshape(2 * N, H)
  adjr = adj_indices.astype(jnp.int32).reshape(2, NCHUNK, J, KB)
  valr = adj_values.astype(jnp.float32)
  idxu = users.astype(jnp.int32).reshape(B // RKB, RKB)
  idxp = (pos_items.astype(jnp.int32) + N_USER).reshape(B // RKB, RKB)
  idxn = (neg_items.astype(jnp.int32) + N_USER).reshape(B // RKB, RKB)
  return _sc_call(ego2, adjr, valr, idxu, idxp, idxn)
